# Initial kernel scaffold; baseline (speedup 1.0000x reference)
import jax
import jax.numpy as jnp
from jax.experimental import pallas as pl


def kernel(x, S, W1, b1, W2, b2):
    raise NotImplementedError("write your pallas kernel here")


# speedup vs baseline: 15.0149x; 15.0149x over previous
import functools

import jax
import jax.numpy as jnp
from jax.experimental import pallas as pl
from jax.experimental.pallas import tpu as pltpu

N = 4096
K = 16
EPS = 0.1
NEG_INF = float("-inf")


def _topk_body(s_ref, vals_ref, idx_ref):
    v = s_ref[...]
    c = v.shape[1]
    rows = jax.lax.broadcasted_iota(jnp.int32, (N, c), 0)
    for k in range(K):
        m = jnp.max(v, axis=0, keepdims=True)
        cand = jnp.where(v == m, rows, N)
        am = jnp.min(cand, axis=0, keepdims=True)
        vals_ref[k : k + 1, :] = m
        idx_ref[k : k + 1, :] = am
        v = jnp.where(rows == am, NEG_INF, v)


def _topk(S):
    C = 256
    grid = (N // C,)
    return pl.pallas_call(
        _topk_body,
        grid=grid,
        in_specs=[pl.BlockSpec((N, C), lambda j: (0, j))],
        out_specs=[
            pl.BlockSpec((K, C), lambda j: (0, j)),
            pl.BlockSpec((K, C), lambda j: (0, j)),
        ],
        out_shape=[
            jax.ShapeDtypeStruct((K, N), jnp.float32),
            jax.ShapeDtypeStruct((K, N), jnp.int32),
        ],
        compiler_params=pltpu.CompilerParams(
            dimension_semantics=("arbitrary",)
        ),
    )(S)


def _build_body(vals_ref, idx_ref, h_ref, degd_ref, binv_ref):
    rb = pl.program_id(0)
    r = h_ref.shape[0]
    mv = (vals_ref[...] > EPS).astype(jnp.float32)
    iv = idx_ref[...]
    rows = jax.lax.broadcasted_iota(jnp.int32, (r, 1), 0) + rb * r
    acc = jnp.zeros((r, N), jnp.float32)
    for k in range(K):
        acc = acc + jnp.where(iv[k : k + 1, :] == rows, mv[k : k + 1, :], 0.0)
    h_ref[...] = acc
    degd_ref[...] = jnp.sum(acc, axis=1, keepdims=True)

    @pl.when(rb == 0)
    def _():
        degb = jnp.sum(mv, axis=0, keepdims=True)
        binv_ref[...] = jnp.where(degb > 0, 1.0 / jnp.maximum(degb, 1e-9), 0.0)


def _build(vals, idx):
    R = 512
    grid = (N // R,)
    return pl.pallas_call(
        _build_body,
        grid=grid,
        in_specs=[
            pl.BlockSpec((K, N), lambda i: (0, 0)),
            pl.BlockSpec((K, N), lambda i: (0, 0)),
        ],
        out_specs=[
            pl.BlockSpec((R, N), lambda i: (i, 0)),
            pl.BlockSpec((R, 1), lambda i: (i, 0)),
            pl.BlockSpec((1, N), lambda i: (0, 0)),
        ],
        out_shape=[
            jax.ShapeDtypeStruct((N, N), jnp.float32),
            jax.ShapeDtypeStruct((N, 1), jnp.float32),
            jax.ShapeDtypeStruct((1, N), jnp.float32),
        ],
        compiler_params=pltpu.CompilerParams(
            dimension_semantics=("arbitrary",)
        ),
    )(vals, idx)


def _agg_t_body(h_ref, x_ref, out_ref):
    kb = pl.program_id(1)
    prod = jax.lax.dot_general(
        h_ref[...], x_ref[...], (((0,), (0,)), ((), ())),
        preferred_element_type=jnp.float32,
    )

    @pl.when(kb == 0)
    def _():
        out_ref[...] = prod

    @pl.when(kb != 0)
    def _():
        out_ref[...] += prod


def _agg_t(H, x):
    F = x.shape[1]
    J = 1024
    R = 1024
    grid = (N // J, N // R)
    return pl.pallas_call(
        _agg_t_body,
        grid=grid,
        in_specs=[
            pl.BlockSpec((R, J), lambda j, k: (k, j)),
            pl.BlockSpec((R, F), lambda j, k: (k, 0)),
        ],
        out_specs=pl.BlockSpec((J, F), lambda j, k: (j, 0)),
        out_shape=jax.ShapeDtypeStruct((N, F), jnp.float32),
        compiler_params=pltpu.CompilerParams(
            dimension_semantics=("parallel", "arbitrary")
        ),
    )(H, x)


def _scatter_body(h_ref, he_ref, binv_ref, degd_ref, bias_ref, out_ref, *,
                  nk, final_tanh):
    kb = pl.program_id(1)
    hb = h_ref[...] * binv_ref[...]
    prod = jnp.dot(hb, he_ref[...], preferred_element_type=jnp.float32)

    @pl.when(kb == 0)
    def _():
        out_ref[...] = prod

    @pl.when(kb != 0)
    def _():
        out_ref[...] += prod

    @pl.when(kb == nk - 1)
    def _():
        dv = degd_ref[...]
        dinv = jnp.where(dv > 0, 1.0 / jnp.maximum(dv, 1e-9), 0.0)
        r = out_ref[...] * dinv + bias_ref[...]
        out_ref[...] = jnp.tanh(r) if final_tanh else r


def _scatter(H, he, binv, degd, bias, final_tanh):
    F = he.shape[1]
    R = 1024
    J = 1024
    nk = N // J
    grid = (N // R, nk)
    return pl.pallas_call(
        functools.partial(_scatter_body, nk=nk, final_tanh=final_tanh),
        grid=grid,
        in_specs=[
            pl.BlockSpec((R, J), lambda i, k: (i, k)),
            pl.BlockSpec((J, F), lambda i, k: (k, 0)),
            pl.BlockSpec((1, J), lambda i, k: (0, k)),
            pl.BlockSpec((R, 1), lambda i, k: (i, 0)),
            pl.BlockSpec((1, F), lambda i, k: (0, 0)),
        ],
        out_specs=pl.BlockSpec((R, F), lambda i, k: (i, 0)),
        out_shape=jax.ShapeDtypeStruct((N, F), jnp.float32),
        compiler_params=pltpu.CompilerParams(
            dimension_semantics=("parallel", "arbitrary")
        ),
    )(H, he, binv, degd, bias)


def _mlp_body(z_ref, w1_ref, b1_ref, w2_ref, out_ref):
    mid = jnp.dot(z_ref[...], w1_ref[...], preferred_element_type=jnp.float32)
    mid = jnp.maximum(mid + b1_ref[...], 0.0)
    out_ref[...] = jnp.dot(mid, w2_ref[...], preferred_element_type=jnp.float32)


def _mlp(z, W1, b1, W2):
    IN_F, HID = W1.shape
    CODE = W2.shape[1]
    R = 512
    grid = (N // R,)
    return pl.pallas_call(
        _mlp_body,
        grid=grid,
        in_specs=[
            pl.BlockSpec((R, IN_F), lambda i: (i, 0)),
            pl.BlockSpec((IN_F, HID), lambda i: (0, 0)),
            pl.BlockSpec((1, HID), lambda i: (0, 0)),
            pl.BlockSpec((HID, CODE), lambda i: (0, 0)),
        ],
        out_specs=pl.BlockSpec((R, CODE), lambda i: (i, 0)),
        out_shape=jax.ShapeDtypeStruct((N, CODE), jnp.float32),
        compiler_params=pltpu.CompilerParams(
            dimension_semantics=("arbitrary",)
        ),
    )(z, W1, b1, W2)


def kernel(x, S, W1, b1, W2, b2):
    vals, idx = _topk(S)
    H, degd, binv = _build(vals, idx)
    zero_b = jnp.zeros((1, x.shape[1]), jnp.float32)
    he = _agg_t(H, x)
    z = _scatter(H, he, binv, degd, zero_b, final_tanh=False)
    t = _mlp(z, W1, b1.reshape(1, -1), W2)
    he2 = _agg_t(H, t)
    code = _scatter(H, he2, binv, degd, b2.reshape(1, -1), final_tanh=True)
    return code
